# baseline (speedup 1.0000x reference)
import functools

import jax
import jax.numpy as jnp
from jax import lax
from jax.experimental import pallas as pl
from jax.experimental.pallas import tpu as pltpu
from jax.experimental.pallas import tpu_sc as plsc

BATCH = 16384
COLS = 4
TOT = BATCH * COLS
NC, NS, L = 2, 16, 16
NW = NC * NS
ROWS_W = BATCH // NW

_mesh = plsc.VectorSubcoreMesh(core_axis_name="c", subcore_axis_name="s")


@functools.partial(
    pl.kernel,
    mesh=_mesh,
    out_type=jax.ShapeDtypeStruct((TOT,), jnp.float32),
    scratch_types=(
        [pltpu.VMEM((ROWS_W,), jnp.int32) for _ in range(COLS)]
        + [pltpu.VMEM((ROWS_W,), jnp.float32) for _ in range(COLS)]
        + [pltpu.VMEM((ROWS_W,), jnp.float32) for _ in range(COLS)]
        + [pltpu.SemaphoreType.DMA((COLS,)),
           pltpu.SemaphoreType.DMA((COLS,)),
           pltpu.SemaphoreType.DMA((COLS,))]
    ),
)
def _gather_sc(xt, embed, s_cm, *refs):
    idx_v = refs[0:COLS]
    val_v = refs[COLS:2 * COLS]
    out_v = refs[2 * COLS:3 * COLS]
    sem_i, sem_g, sem_o = refs[3 * COLS:]

    wid = lax.axis_index("s") * NC + lax.axis_index("c")
    rbase = wid * ROWS_W

    idx_cp = [
        pltpu.async_copy(xt.at[pl.ds(c * BATCH + rbase, ROWS_W)], idx_v[c],
                         sem_i.at[c])
        for c in range(COLS)
    ]
    gathers = []
    for c in range(COLS):
        idx_cp[c].wait()
        gathers.append(
            pltpu.async_copy(embed.at[idx_v[c]], val_v[c], sem_g.at[c]))
    for g in gathers:
        g.wait()

    def body(i, carry):
        sl = pl.ds(i * L, L)
        v = [val_v[c][sl] for c in range(COLS)]
        t = (v[0] + v[1]) + (v[2] + v[3])
        for c in range(COLS):
            out_v[c][sl] = 4.0 * v[c] - t
        return carry

    lax.fori_loop(0, ROWS_W // L, body, 0)
    out_cp = [
        pltpu.async_copy(out_v[c], s_cm.at[pl.ds(c * BATCH + rbase, ROWS_W)],
                         sem_o.at[c])
        for c in range(COLS)
    ]
    for cp in out_cp:
        cp.wait()


def kernel(X, embed):
    xt = X.T.reshape(TOT)
    ef = embed.reshape(embed.shape[0])
    s_cm = _gather_sc(xt, ef)
    return s_cm.reshape(COLS, BATCH).T
